# baseline (speedup 1.0000x reference)
import jax
import jax.numpy as jnp
from jax.experimental import pallas as pl
from jax.experimental.pallas import tpu as pltpu

_H = 128
_NA = 3


def _dot(x, w):
    return jax.lax.dot_general(
        x, w, (((1,), (0,)), ((), ())), preferred_element_type=jnp.float32
    )


def _fused_kernel(s0_ref, s1_ref, s2_ref, a0_ref, a1_ref, a2_ref,
                  W1_ref, b1_ref, Wg_ref, bg_ref, W2_ref, b2_ref,
                  Wha_ref, bha_ref, Whb_ref, bhb_ref, q1_ref, q2_ref):
    x = jnp.concatenate(
        [s0_ref[0], s1_ref[0], s2_ref[0], a0_ref[0], a1_ref[0], a2_ref[0]],
        axis=1,
    )
    fc1 = jnp.maximum(_dot(x, W1_ref[...]) + b1_ref[...], 0.0)

    xw = _dot(fc1, Wg_ref[...])
    m = (xw[0:1, :] + xw[1:2, :] + xw[2:3, :]) * (1.0 / 3.0)
    rows = jax.lax.broadcasted_iota(jnp.int32, (xw.shape[0], 1), 0)
    is_first = pl.program_id(0) == 0
    xw = jnp.where(jnp.logical_and(is_first, rows < _NA), m, xw)

    g = jnp.maximum(xw + bg_ref[...], 0.0) + fc1
    x2 = jnp.maximum(_dot(g, W2_ref[...]) + b2_ref[...], 0.0)

    h = jnp.maximum(_dot(x2, Wha_ref[...]) + bha_ref[...], 0.0)
    q = _dot(h, Whb_ref[...]) + bhb_ref[...]
    q1_ref[...] = q[:, 0:1]
    q2_ref[...] = q[:, 1:2]


def kernel(s, a, W1, b1, Wg, bg, W2, b2, Wq1a, bq1a, Wq1b, bq1b, Wq2a, bq2a,
           Wq2b, bq2b):
    n = s.shape[1]
    obs = s.shape[2]
    act = a.shape[2]

    block = 4000 if n % 4000 == 0 else n
    grid = n // block

    Wha = jnp.concatenate([Wq1a, Wq2a], axis=1)
    bha = jnp.concatenate([bq1a, bq2a], axis=0).reshape(1, 2 * _H)
    Whb = jnp.concatenate(
        [
            jnp.concatenate([Wq1b, jnp.zeros_like(Wq1b)], axis=1),
            jnp.concatenate([jnp.zeros_like(Wq2b), Wq2b], axis=1),
        ],
        axis=0,
    )
    bhb = jnp.concatenate([bq1b, bq2b], axis=0).reshape(1, 2)

    b1r = b1.reshape(1, _H)
    bgr = bg.reshape(1, _H)
    b2r = b2.reshape(1, _H)

    wspec = pl.BlockSpec(lambda i: (0, 0))

    q = pl.pallas_call(
        _fused_kernel,
        grid=(grid,),
        in_specs=[
            pl.BlockSpec((1, block, obs), lambda i: (0, i, 0)),
            pl.BlockSpec((1, block, obs), lambda i: (1, i, 0)),
            pl.BlockSpec((1, block, obs), lambda i: (2, i, 0)),
            pl.BlockSpec((1, block, act), lambda i: (0, i, 0)),
            pl.BlockSpec((1, block, act), lambda i: (1, i, 0)),
            pl.BlockSpec((1, block, act), lambda i: (2, i, 0)),
            pl.BlockSpec(W1.shape, lambda i: (0, 0)),
            pl.BlockSpec((1, _H), lambda i: (0, 0)),
            pl.BlockSpec(Wg.shape, lambda i: (0, 0)),
            pl.BlockSpec((1, _H), lambda i: (0, 0)),
            pl.BlockSpec(W2.shape, lambda i: (0, 0)),
            pl.BlockSpec((1, _H), lambda i: (0, 0)),
            pl.BlockSpec((_H, 2 * _H), lambda i: (0, 0)),
            pl.BlockSpec((1, 2 * _H), lambda i: (0, 0)),
            pl.BlockSpec((2 * _H, 2), lambda i: (0, 0)),
            pl.BlockSpec((1, 2), lambda i: (0, 0)),
        ],
        out_specs=[
            pl.BlockSpec((block, 1), lambda i: (i, 0)),
            pl.BlockSpec((block, 1), lambda i: (i, 0)),
        ],
        out_shape=[
            jax.ShapeDtypeStruct((n, 1), jnp.float32),
            jax.ShapeDtypeStruct((n, 1), jnp.float32),
        ],
        compiler_params=pltpu.CompilerParams(
            dimension_semantics=("parallel",),
        ),
    )(s, s, s, a, a, a, W1, b1r, Wg, bgr, W2, b2r, Wha, bha, Whb, bhb)

    return (q[0], q[1])


# speedup vs baseline: 1.8959x; 1.8959x over previous
import jax
import jax.numpy as jnp
from jax.experimental import pallas as pl
from jax.experimental.pallas import tpu as pltpu

_H = 128
_NA = 3


def _dott(w, x):
    return jax.lax.dot_general(
        w, x, (((1,), (0,)), ((), ())), preferred_element_type=jnp.float32
    )


def _fused_kernel(x_ref, W1t_ref, b1_ref, Wgt_ref, bg_ref, W2t_ref, b2_ref,
                  What_ref, bha_ref, Whbt_ref, bhb_ref, q1_ref, q2_ref):
    x = x_ref[...]
    fc1 = jnp.maximum(_dott(W1t_ref[...], x) + b1_ref[...], 0.0)

    xw = _dott(Wgt_ref[...], fc1)
    m = (xw[:, 0:1] + xw[:, 1:2] + xw[:, 2:3]) * (1.0 / 3.0)
    cols = jax.lax.broadcasted_iota(jnp.int32, (1, xw.shape[1]), 1)
    is_first = pl.program_id(0) == 0
    xw = jnp.where(jnp.logical_and(is_first, cols < _NA), m, xw)

    g = jnp.maximum(xw + bg_ref[...], 0.0) + fc1
    x2 = jnp.maximum(_dott(W2t_ref[...], g) + b2_ref[...], 0.0)

    h = jnp.maximum(_dott(What_ref[...], x2) + bha_ref[...], 0.0)
    q = _dott(Whbt_ref[...], h) + bhb_ref[...]
    qt = jnp.swapaxes(q, 0, 1)
    q1_ref[...] = qt[:, 0:1]
    q2_ref[...] = qt[:, 1:2]


def kernel(s, a, W1, b1, Wg, bg, W2, b2, Wq1a, bq1a, Wq1b, bq1b, Wq2a, bq2a,
           Wq2b, bq2b):
    n = s.shape[1]
    obs = s.shape[2]
    act = a.shape[2]
    in_dim = _NA * (obs + act)

    block = 8192
    grid = (n + block - 1) // block

    xt = jnp.concatenate(
        [
            s.transpose(0, 2, 1).reshape(_NA * obs, n),
            a.transpose(0, 2, 1).reshape(_NA * act, n),
        ],
        axis=0,
    )

    W1t = W1.T
    Wgt = Wg.T
    W2t = W2.T
    What = jnp.concatenate([Wq1a, Wq2a], axis=1).T
    bha = jnp.concatenate([bq1a, bq2a], axis=0).reshape(2 * _H, 1)
    Whbt = jnp.concatenate(
        [
            jnp.concatenate([Wq1b, jnp.zeros_like(Wq1b)], axis=1),
            jnp.concatenate([jnp.zeros_like(Wq2b), Wq2b], axis=1),
        ],
        axis=0,
    ).T
    bhb = jnp.concatenate([bq1b, bq2b], axis=0).reshape(2, 1)

    b1r = b1.reshape(_H, 1)
    bgr = bg.reshape(_H, 1)
    b2r = b2.reshape(_H, 1)

    q1, q2 = pl.pallas_call(
        _fused_kernel,
        grid=(grid,),
        in_specs=[
            pl.BlockSpec((in_dim, block), lambda i: (0, i)),
            pl.BlockSpec((_H, in_dim), lambda i: (0, 0)),
            pl.BlockSpec((_H, 1), lambda i: (0, 0)),
            pl.BlockSpec((_H, _H), lambda i: (0, 0)),
            pl.BlockSpec((_H, 1), lambda i: (0, 0)),
            pl.BlockSpec((_H, _H), lambda i: (0, 0)),
            pl.BlockSpec((_H, 1), lambda i: (0, 0)),
            pl.BlockSpec((2 * _H, _H), lambda i: (0, 0)),
            pl.BlockSpec((2 * _H, 1), lambda i: (0, 0)),
            pl.BlockSpec((2, 2 * _H), lambda i: (0, 0)),
            pl.BlockSpec((2, 1), lambda i: (0, 0)),
        ],
        out_specs=[
            pl.BlockSpec((block, 1), lambda i: (i, 0)),
            pl.BlockSpec((block, 1), lambda i: (i, 0)),
        ],
        out_shape=[
            jax.ShapeDtypeStruct((n, 1), jnp.float32),
            jax.ShapeDtypeStruct((n, 1), jnp.float32),
        ],
        compiler_params=pltpu.CompilerParams(
            dimension_semantics=("parallel",),
            vmem_limit_bytes=100 * 1024 * 1024,
        ),
    )(xt, W1t, b1r, Wgt, bgr, W2t, b2r, What, bha, Whbt, bhb)

    return (q1, q2)
